# Initial kernel scaffold; baseline (speedup 1.0000x reference)
import jax
import jax.numpy as jnp
from jax.experimental import pallas as pl


def kernel(x, adj_list, W0, W1, b, Wp, bp):
    raise NotImplementedError("write your pallas kernel here")


# speedup vs baseline: 6900.3422x; 6900.3422x over previous
import jax
import jax.numpy as jnp
from jax.experimental import pallas as pl
from jax.experimental.pallas import tpu as pltpu


def _body(adj_ref, x_ref, w0_ref, w1_ref, b_ref, wp_ref, bp_ref, out_ref):
    i = pl.program_id(0)
    adj = adj_ref[0]
    xv = x_ref[...]
    deg = jnp.sum(adj, axis=1, keepdims=True)
    dis = jnp.where(deg > 0, jax.lax.rsqrt(deg), 0.0)
    y = dis * xv
    z = jnp.dot(adj, y, preferred_element_type=jnp.float32)
    tx1 = -(dis * z)
    o = (jnp.dot(xv, w0_ref[0].T, preferred_element_type=jnp.float32)
         + jnp.dot(tx1, w1_ref[0].T, preferred_element_type=jnp.float32)
         + b_ref[0])
    contrib = jnp.dot(o, wp_ref[...].T, preferred_element_type=jnp.float32)

    @pl.when(i == 0)
    def _init():
        out_ref[...] = contrib + bp_ref[...]

    @pl.when(i != 0)
    def _acc():
        out_ref[...] += contrib


def kernel(x, adj_list, W0, W1, b, Wp, bp):
    B, N, C = x.shape
    V = adj_list.shape[0]
    OUT = W0.shape[1]
    x2 = x.reshape(N, C)
    b3 = b.reshape(V, 1, OUT)
    bp2 = bp.reshape(1, OUT)

    out = pl.pallas_call(
        _body,
        grid=(V,),
        in_specs=[
            pl.BlockSpec((1, N, N), lambda i: (i, 0, 0)),
            pl.BlockSpec((N, C), lambda i: (0, 0)),
            pl.BlockSpec((1, OUT, C), lambda i: (i, 0, 0)),
            pl.BlockSpec((1, OUT, C), lambda i: (i, 0, 0)),
            pl.BlockSpec((1, 1, OUT), lambda i: (i, 0, 0)),
            pl.BlockSpec((OUT, OUT), lambda i: (0, i)),
            pl.BlockSpec((1, OUT), lambda i: (0, 0)),
        ],
        out_specs=pl.BlockSpec((N, OUT), lambda i: (0, 0)),
        out_shape=jax.ShapeDtypeStruct((N, OUT), jnp.float32),
        compiler_params=pltpu.CompilerParams(
            dimension_semantics=("arbitrary",),
        ),
    )(adj_list, x2, W0, W1, b3, Wp, bp2)
    return out.reshape(B, N, OUT)
